# baseline (speedup 1.0000x reference)
import jax
import jax.numpy as jnp
from jax import lax
from jax.experimental import pallas as pl
from jax.experimental.pallas import tpu as pltpu
from jax.experimental.pallas import tpu_sc as plsc

M = 200000
N = 4000000
KC = 100000
KG = 2000000

NW = 32

SCAN_CHUNK = 2000

MB_CTL = 3
MB_GAU = 61

_IOTA = lambda: lax.iota(jnp.int32, 16)


def _scatter_pass(w, idx_hbm, new_hbm, in_hbm, out_hbm, ticket, idxb, inbuf,
                  vals, klist, klist3, plist, cnts, scan_sem, in_sem, g_sem,
                  out_sem, *, n_chunks, base_rows_std, rows_std, rows_last,
                  strip):
    is_last = w == NW - 1
    nrows = jnp.where(is_last, rows_last, rows_std)
    base = base_rows_std + w * rows_std
    nstrips = nrows // strip
    vpc = SCAN_CHUNK // 16
    vps = strip // 16

    zero16 = jnp.zeros((16,), jnp.int32)

    def _clr(i, _):
        ticket[pl.ds(32 * i, 16)] = zero16
        ticket[pl.ds(32 * i + 16, 16)] = zero16
        return 0

    lax.fori_loop(0, nrows // 32, _clr, 0)

    nvec = jnp.full((16,), nrows, jnp.int32)
    basev = jnp.full((16,), base, jnp.int32)
    iota = _IOTA()

    def _chunk_dma(c, slot):
        return pltpu.make_async_copy(
            idx_hbm.at[pl.ds(c * SCAN_CHUNK, SCAN_CHUNK)],
            idxb.at[pl.ds(SCAN_CHUNK * slot, SCAN_CHUNK)], scan_sem.at[slot])

    _chunk_dma(0, 0).start()
    _chunk_dma(1, 1).start()

    def _do_chunk(c, slot):
        _chunk_dma(c, slot).wait()
        kb = c * SCAN_CHUNK + 1
        kv0 = jnp.full((16,), kb, jnp.int32) + iota

        def _vstep(i, _):
            for u in range(5):
                v = idxb[pl.ds(SCAN_CHUNK * slot + 16 * (5 * i + u), 16)]
                local = v - basev
                inb = (local >= 0) & (local < nvec)
                lcl = jnp.minimum(jnp.maximum(local, 0), nvec - 1)
                kv = kv0 + (80 * i + 16 * u)
                plsc.store_scatter(ticket, [lcl], kv, mask=inb)
            return 0

        lax.fori_loop(0, SCAN_CHUNK // 80, _vstep, 0)
        @pl.when(c + 2 < n_chunks)
        def _():
            _chunk_dma(c + 2, slot).start()

    def _scan_pair(gp, _):
        _do_chunk(2 * gp, 0)
        _do_chunk(2 * gp + 1, 1)
        return 0

    lax.fori_loop(0, n_chunks // 2, _scan_pair, 0)

    padv = jnp.full((16,), w * 1024, jnp.int32) + iota
    sbr = 8 * strip
    sbe = 3 * sbr
    vpsb = sbr // 16

    def _in_dma(s, p):
        return pltpu.make_async_copy(
            in_hbm.at[pl.ds(3 * (base + s * sbr), sbe)],
            inbuf.at[pl.ds(3072 * p, sbe)], in_sem.at[p])

    def _out_dma(s, p):
        return pltpu.make_async_copy(
            inbuf.at[pl.ds(3072 * p, sbe)],
            out_hbm.at[pl.ds(3 * (base + s * sbr), sbe)], out_sem.at[p])

    def _g_dma(p, c):
        return pltpu.make_async_copy(
            new_hbm.at[klist3.at[pl.ds(3072 * p + 1024 * c, sbr)]],
            vals.at[pl.ds(3072 * p + 1024 * c, sbr)], g_sem.at[p])

    def _extract(s, p):
        def _pre(j, _):
            klist[pl.ds(1024 * p + 16 * j, 16)] = padv + 16 * j
            return 0

        lax.fori_loop(0, vpsb, _pre, 0)
        srow = s * sbr

        def _ex(j, off):
            t = ticket[pl.ds(srow + 16 * j, 16)]
            msk = t > 0
            plsc.store_compressed(klist.at[pl.ds(1024 * p + off, 16)],
                                  t - 1, mask=msk)
            plsc.store_compressed(plist.at[pl.ds(1024 * p + off, 16)],
                                  iota + 16 * j, mask=msk)
            return off + jnp.sum(msk.astype(jnp.int32))

        cnts[p] = lax.fori_loop(0, vpsb, _ex, jnp.int32(0))

        def _exp(j, _):
            kk = klist[pl.ds(1024 * p + 16 * j, 16)]
            k3 = kk * 3
            for c in range(3):
                klist3[pl.ds(3072 * p + 1024 * c + 16 * j, 16)] = k3 + c
            return 0

        lax.fori_loop(0, vpsb, _exp, 0)

    def _gather(p):
        for c in range(3):
            _g_dma(p, c).start()

    def _gwait(p):
        for c in range(3):
            _g_dma(p, c).wait()

    def _patch(p):
        cnt = cnts[p]
        vb = jnp.full((16,), 3072 * p, jnp.int32)
        lb = jnp.full((16,), 1024 * p, jnp.int32)

        def _pstep(t, _):
            jv = iota + 16 * t
            mv = jv < cnt
            jc = jnp.minimum(jv, sbr - 1)
            pos = plsc.load_gather(plist, [lb + jc], mask=mv)
            posc = jnp.minimum(jnp.maximum(pos, 0), sbr - 1)
            p3 = posc * 3
            for c in range(3):
                x = plsc.load_gather(vals, [vb + 1024 * c + jc], mask=mv)
                plsc.store_scatter(inbuf, [vb + p3 + c], x, mask=mv)
            return 0

        lax.fori_loop(0, (cnt + 15) // 16, _pstep, 0)

    nsb = nstrips // 8
    rem = nstrips - nsb * 8

    @pl.when(nsb > 0)
    def _():
        _in_dma(0, 0).start()

    def _sb_pair(q, _):
        for p in range(2):
            s = 2 * q + p

            @pl.when(s < nsb)
            def _():
                _in_dma(s, p).wait()
                _extract(s, p)
                _gather(p)
                @pl.when((s + 1 < nsb) & (s >= 1))
                def _():
                    _out_dma(s - 1, 1 - p).wait()
                @pl.when(s + 1 < nsb)
                def _():
                    _in_dma(s + 1, 1 - p).start()
                _gwait(p)
                _patch(p)
                _out_dma(s, p).start()
        return 0

    lax.fori_loop(0, (nsb + 1) // 2, _sb_pair, 0)
    for p in range(2):
        @pl.when(nsb >= p + 1)
        def _():
            _out_dma(0, p).wait()

    def _tail(j, _):
        s0 = nsb * 8 + j
        cp_in = pltpu.make_async_copy(
            in_hbm.at[pl.ds(3 * (base + s0 * strip), 3 * strip)],
            inbuf.at[pl.ds(0, 3 * strip)], in_sem.at[0])
        cp_in.start()
        cp_in.wait()

        def _pre(jx, _):
            klist[pl.ds(16 * jx, 16)] = padv + 16 * jx
            return 0

        lax.fori_loop(0, strip // 16, _pre, 0)
        srow = s0 * strip

        def _ex(jx, off):
            t = ticket[pl.ds(srow + 16 * jx, 16)]
            msk = t > 0
            plsc.store_compressed(klist.at[pl.ds(off, 16)], t - 1, mask=msk)
            plsc.store_compressed(plist.at[pl.ds(off, 16)], iota + 16 * jx,
                                  mask=msk)
            return off + jnp.sum(msk.astype(jnp.int32))

        cnt = lax.fori_loop(0, strip // 16, _ex, jnp.int32(0))

        def _exp(jx, _):
            kk = klist[pl.ds(16 * jx, 16)]
            k3 = kk * 3
            for c in range(3):
                klist3[pl.ds(1024 * c + 16 * jx, 16)] = k3 + c
            return 0

        lax.fori_loop(0, strip // 16, _exp, 0)
        for c in range(3):
            pltpu.make_async_copy(
                new_hbm.at[klist3.at[pl.ds(1024 * c, strip)]],
                vals.at[pl.ds(1024 * c, strip)], g_sem.at[0]).start()
        for c in range(3):
            pltpu.make_async_copy(
                new_hbm.at[klist3.at[pl.ds(1024 * c, strip)]],
                vals.at[pl.ds(1024 * c, strip)], g_sem.at[0]).wait()

        def _pstep(t, _):
            jv = iota + 16 * t
            mv = jv < cnt
            jc = jnp.minimum(jv, strip - 1)
            pos = plsc.load_gather(plist, [jc], mask=mv)
            posc = jnp.minimum(jnp.maximum(pos, 0), strip - 1)
            p3 = posc * 3
            for c in range(3):
                x = plsc.load_gather(vals, [1024 * c + jc], mask=mv)
                plsc.store_scatter(inbuf, [p3 + c], x, mask=mv)
            return 0

        lax.fori_loop(0, (cnt + 15) // 16, _pstep, 0)
        cp_out = pltpu.make_async_copy(
            inbuf.at[pl.ds(0, 3 * strip)],
            out_hbm.at[pl.ds(3 * (base + s0 * strip), 3 * strip)],
            out_sem.at[0])
        cp_out.start()
        cp_out.wait()
        return 0

    lax.fori_loop(0, rem, _tail, 0)


def _scatter_body(ctl_in, gau_in, new_ctl, new_gau, ci, gi,
                  ctl_out, gau_out, ticket, idxb, inbuf, vals, klist, klist3,
                  plist, cnts, scan_sem, in_sem, g_sem, out_sem):
    w = lax.axis_index("s") * 2 + lax.axis_index("c")
    common = (ticket, idxb, inbuf, vals, klist, klist3, plist, cnts,
              scan_sem, in_sem, g_sem, out_sem)
    _scatter_pass(w, ci, new_ctl, ctl_in, ctl_out, *common,
                  n_chunks=KC // SCAN_CHUNK, base_rows_std=0,
                  rows_std=6240, rows_last=6560, strip=32)
    def _gau_half(h, _):
        _scatter_pass(w, gi, new_gau, gau_in, gau_out, *common,
                      n_chunks=KG // SCAN_CHUNK,
                      base_rows_std=h * 2000000,
                      rows_std=62464, rows_last=63616, strip=128)
        return 0

    lax.fori_loop(0, 2, _gau_half, 0)


def _mean_table(w, table_hbm, idx_hbm, stg, rows, idx3, outv, stg_sem, g_sem,
                *, nblk, extra128, extra32, out_off, partials):
    ibase = w * (nblk * 1024)
    rem_base = NW * (nblk * 1024)
    iota = _IOTA()
    accs = [jnp.zeros((16,), jnp.float32) for _ in range(3)]

    def _stg_dma(blk, slot, size):
        return pltpu.make_async_copy(
            idx_hbm.at[pl.ds(ibase + blk * 1024, size)],
            stg.at[pl.ds(1024 * slot, size)], stg_sem.at[slot])

    def _expand(jj, stg_slot):
        for r in range(8):
            kk = stg[pl.ds(1024 * stg_slot + 128 * jj + 16 * r, 16)]
            k3 = kk * 3
            for c in range(3):
                idx3[pl.ds(384 * jj + 128 * c + 16 * r, 16)] = k3 + c

    def _g_dma(jj, c):
        return pltpu.make_async_copy(
            table_hbm.at[idx3.at[pl.ds(384 * jj + 128 * c, 128)]],
            rows.at[pl.ds(384 * jj + 128 * c, 128)], g_sem.at[jj])

    def _fire(jj, stg_slot):
        _expand(jj, stg_slot)
        for c in range(3):
            _g_dma(jj, c).start()

    def _drain(jj):
        for c in range(3):
            _g_dma(jj, c).wait()

    def _acc_chunk(accs, jj):
        for rv in range(8):
            for c in range(3):
                x = rows[pl.ds(384 * jj + 128 * c + 16 * rv, 16)]
                accs[c] = accs[c] + x
        return accs

    _stg_dma(0, 0, 1024).start()
    _stg_dma(0, 0, 1024).wait()
    for jj in range(8):
        _fire(jj, 0)
    _stg_dma(1, 1, 1024).start()

    def _pair(bp, accs):
        accs = list(accs)
        blk_e = 2 * bp
        _stg_dma(blk_e + 1, 1, 1024).wait()
        for jj in range(8):
            _drain(jj)
            accs = _acc_chunk(accs, jj)
            _fire(jj, 1)
        @pl.when(blk_e + 2 < nblk)
        def _():
            _stg_dma(blk_e + 2, 0, 1024).start()
            _stg_dma(blk_e + 2, 0, 1024).wait()
        @pl.when(blk_e + 3 < nblk)
        def _():
            _stg_dma(blk_e + 3, 1, 1024).start()
        for jj in range(8):
            _drain(jj)
            accs = _acc_chunk(accs, jj)
            @pl.when(blk_e + 2 < nblk)
            def _():
                _fire(jj, 0)
        return tuple(accs)

    accs = list(lax.fori_loop(0, nblk // 2, _pair, tuple(accs)))
    for jj in range(8):
        _drain(jj)
        accs = _acc_chunk(accs, jj)

    sums = [jnp.sum(a) for a in accs]
    vec = jnp.where(iota == 0, jnp.full((16,), sums[0], jnp.float32),
          jnp.where(iota == 1, jnp.full((16,), sums[1], jnp.float32),
          jnp.where(iota == 2, jnp.full((16,), sums[2], jnp.float32),
                    jnp.zeros((16,), jnp.float32))))
    outv[...] = vec

    @pl.when(w == NW - 1)
    def _():
        exaccs = [jnp.zeros((16,), jnp.float32) for _ in range(3)]

        def _extra_body(j, carry):
            cp = pltpu.make_async_copy(
                idx_hbm.at[pl.ds(rem_base + j * 128, 128)],
                stg.at[pl.ds(0, 128)], stg_sem.at[0])
            cp.start()
            cp.wait()
            _fire(0, 0)
            _drain(0)
            return tuple(_acc_chunk(list(carry), 0))

        ex = list(lax.fori_loop(0, extra128, _extra_body, tuple(exaccs)))
        if extra32:
            base32 = rem_base + extra128 * 128
            cp = pltpu.make_async_copy(
                idx_hbm.at[pl.ds(base32, 32)], stg.at[pl.ds(0, 32)],
                stg_sem.at[0])
            cp.start()
            cp.wait()
            for r in range(2):
                kk = stg[pl.ds(16 * r, 16)]
                k3 = kk * 3
                for c in range(3):
                    idx3[pl.ds(32 * c + 16 * r, 16)] = k3 + c
            for c in range(3):
                pltpu.make_async_copy(
                    table_hbm.at[idx3.at[pl.ds(32 * c, 32)]],
                    rows.at[pl.ds(32 * c, 32)], g_sem.at[0]).start()
            for c in range(3):
                pltpu.make_async_copy(
                    table_hbm.at[idx3.at[pl.ds(32 * c, 32)]],
                    rows.at[pl.ds(32 * c, 32)], g_sem.at[0]).wait()
            for rv in range(2):
                for c in range(3):
                    x = rows[pl.ds(32 * c + 16 * rv, 16)]
                    ex[c] = ex[c] + x
        exsums = [jnp.sum(a) for a in ex]
        exvec = jnp.where(iota == 0, jnp.full((16,), exsums[0], jnp.float32),
                jnp.where(iota == 1, jnp.full((16,), exsums[1], jnp.float32),
                jnp.where(iota == 2, jnp.full((16,), exsums[2], jnp.float32),
                          jnp.zeros((16,), jnp.float32))))
        outv[...] = outv[...] + exvec

    pltpu.sync_copy(outv, partials.at[w, pl.ds(out_off, 16)])


def _mean_body(ctl_in, gau_in, ci, gi, partials, stg, rows, idx3,
               outv, stg_sem, g_sem):
    w = lax.axis_index("s") * 2 + lax.axis_index("c")
    _mean_table(w, ctl_in, ci, stg, rows, idx3, outv, stg_sem, g_sem,
                nblk=MB_CTL, extra128=13, extra32=True, out_off=0,
                partials=partials)
    _mean_table(w, gau_in, gi, stg, rows, idx3, outv, stg_sem, g_sem,
                nblk=MB_GAU, extra128=9, extra32=False, out_off=16,
                partials=partials)


def _reduce_body(p_ref, o_ref):
    s = jnp.sum(p_ref[...], axis=0, keepdims=True)
    scale = jnp.concatenate([
        jnp.full((1, 3), 0.5 / KC, jnp.float32),
        jnp.zeros((1, 13), jnp.float32),
        jnp.full((1, 3), 0.5 / KG, jnp.float32),
        jnp.zeros((1, 13), jnp.float32),
    ], axis=1)
    o_ref[...] = jnp.pad(s * scale, ((0, 7), (0, 96)))


@jax.jit
def kernel(control_xyz, gaussian_xyz, new_control_xyz, new_gaussian_xyz,
           control_indices, gaussian_indices):
    rt0 = control_indices[0].astype(jnp.float32) * 0.0
    ctl_flat = control_xyz.reshape(-1) + rt0
    gau_flat = gaussian_xyz.reshape(-1) + rt0
    new_ctl_flat = new_control_xyz.reshape(-1) + rt0
    new_gau_flat = new_gaussian_xyz.reshape(-1) + rt0

    mesh = plsc.VectorSubcoreMesh(core_axis_name="c", subcore_axis_name="s")

    scatter_fn = pl.kernel(
        _scatter_body,
        out_type=[
            jax.ShapeDtypeStruct((3 * M,), jnp.float32),
            jax.ShapeDtypeStruct((3 * N,), jnp.float32),
        ],
        mesh=mesh,
        compiler_params=pltpu.CompilerParams(needs_layout_passes=False),
        scratch_types=[
            pltpu.VMEM((63616,), jnp.int32),
            pltpu.VMEM((2 * SCAN_CHUNK,), jnp.int32),
            pltpu.VMEM((2 * 3072,), jnp.float32),
            pltpu.VMEM((2 * 3072,), jnp.float32),
            pltpu.VMEM((2 * 1024,), jnp.int32),
            pltpu.VMEM((2 * 3072,), jnp.int32),
            pltpu.VMEM((2 * 1024,), jnp.int32),
            pltpu.SMEM((2,), jnp.int32),
            pltpu.SemaphoreType.DMA((2,)),
            pltpu.SemaphoreType.DMA((2,)),
            pltpu.SemaphoreType.DMA((2,)),
            pltpu.SemaphoreType.DMA((2,)),
        ],
    )
    updated_ctl_flat, updated_gau_flat = scatter_fn(
        ctl_flat, gau_flat, new_ctl_flat, new_gau_flat, control_indices,
        gaussian_indices)

    mean_fn = pl.kernel(
        _mean_body,
        out_type=jax.ShapeDtypeStruct((NW, 32), jnp.float32),
        mesh=mesh,
        compiler_params=pltpu.CompilerParams(needs_layout_passes=False),
        scratch_types=[
            pltpu.VMEM((2 * 1024,), jnp.int32),
            pltpu.VMEM((8 * 384,), jnp.float32),
            pltpu.VMEM((8 * 384,), jnp.int32),
            pltpu.VMEM((16,), jnp.float32),
            pltpu.SemaphoreType.DMA((2,)),
            pltpu.SemaphoreType.DMA((8,)),
        ],
    )
    partials = mean_fn(ctl_flat, gau_flat, control_indices,
                       gaussian_indices)

    red = pl.pallas_call(
        _reduce_body,
        out_shape=jax.ShapeDtypeStruct((8, 128), jnp.float32),
    )(partials)
    center = red[0, 0:3] + red[0, 16:19]

    return (center, updated_ctl_flat.reshape(M, 3) + 0.0,
            updated_gau_flat.reshape(N, 3) + 0.0)


# speedup vs baseline: 1.0052x; 1.0052x over previous
import jax
import jax.numpy as jnp
from jax import lax
from jax.experimental import pallas as pl
from jax.experimental.pallas import tpu as pltpu
from jax.experimental.pallas import tpu_sc as plsc

M = 200000
N = 4000000
KC = 100000
KG = 2000000

NW = 32

SCAN_CHUNK = 2000

MB_CTL = 3
MB_GAU = 61

_IOTA = lambda: lax.iota(jnp.int32, 16)


def _scatter_pass(w, idx_hbm, new_hbm, in_hbm, out_hbm, ticket, idxb, inbuf,
                  vals, klist, klist3, plist, cnts, scan_sem, in_sem, g_sem,
                  out_sem, *, n_chunks, base_rows_std, rows_std, rows_last,
                  strip):
    is_last = w == NW - 1
    nrows = jnp.where(is_last, rows_last, rows_std)
    base = base_rows_std + w * rows_std
    nstrips = nrows // strip
    vpc = SCAN_CHUNK // 16
    vps = strip // 16

    zero16 = jnp.zeros((16,), jnp.int32)

    def _clr(i, _):
        ticket[pl.ds(32 * i, 16)] = zero16
        ticket[pl.ds(32 * i + 16, 16)] = zero16
        return 0

    lax.fori_loop(0, nrows // 32, _clr, 0)

    nvec = jnp.full((16,), nrows, jnp.int32)
    basev = jnp.full((16,), base, jnp.int32)
    iota = _IOTA()

    def _chunk_dma(c, slot):
        return pltpu.make_async_copy(
            idx_hbm.at[pl.ds(c * SCAN_CHUNK, SCAN_CHUNK)],
            idxb.at[pl.ds(SCAN_CHUNK * slot, SCAN_CHUNK)], scan_sem.at[slot])

    _chunk_dma(0, 0).start()
    _chunk_dma(1, 1).start()

    def _do_chunk(c, slot):
        _chunk_dma(c, slot).wait()
        kb = c * SCAN_CHUNK + 1
        kv0 = jnp.full((16,), kb, jnp.int32) + iota

        def _vstep(i, _):
            for u in range(5):
                v = idxb[pl.ds(SCAN_CHUNK * slot + 16 * (5 * i + u), 16)]
                local = v - basev
                inb = (local >= 0) & (local < nvec)
                lcl = jnp.minimum(jnp.maximum(local, 0), nvec - 1)
                kv = kv0 + (80 * i + 16 * u)
                plsc.store_scatter(ticket, [lcl], kv, mask=inb)
            return 0

        lax.fori_loop(0, SCAN_CHUNK // 80, _vstep, 0)
        @pl.when(c + 2 < n_chunks)
        def _():
            _chunk_dma(c + 2, slot).start()

    def _scan_pair(gp, _):
        _do_chunk(2 * gp, 0)
        _do_chunk(2 * gp + 1, 1)
        return 0

    lax.fori_loop(0, n_chunks // 2, _scan_pair, 0)

    padv = jnp.full((16,), w * 1024, jnp.int32) + iota
    sbr = 8 * strip
    sbe = 3 * sbr
    vpsb = sbr // 16

    def _in_dma(s, p):
        return pltpu.make_async_copy(
            in_hbm.at[pl.ds(3 * (base + s * sbr), sbe)],
            inbuf.at[pl.ds(3072 * p, sbe)], in_sem.at[p])

    def _out_dma(s, p):
        return pltpu.make_async_copy(
            inbuf.at[pl.ds(3072 * p, sbe)],
            out_hbm.at[pl.ds(3 * (base + s * sbr), sbe)], out_sem.at[p])

    def _g_dma(p, c):
        return pltpu.make_async_copy(
            new_hbm.at[klist3.at[pl.ds(3072 * p + 1024 * c, sbr)]],
            vals.at[pl.ds(3072 * p + 1024 * c, sbr)], g_sem.at[p])

    def _extract(s, p):
        def _pre(j, _):
            klist[pl.ds(1024 * p + 16 * j, 16)] = padv + 16 * j
            return 0

        lax.fori_loop(0, vpsb, _pre, 0)
        srow = s * sbr

        def _ex(j, off):
            t = ticket[pl.ds(srow + 16 * j, 16)]
            msk = t > 0
            plsc.store_compressed(klist.at[pl.ds(1024 * p + off, 16)],
                                  t - 1, mask=msk)
            plsc.store_compressed(plist.at[pl.ds(1024 * p + off, 16)],
                                  iota + 16 * j, mask=msk)
            return off + jnp.sum(msk.astype(jnp.int32))

        cnts[p] = lax.fori_loop(0, vpsb, _ex, jnp.int32(0))

        def _exp(j, _):
            kk = klist[pl.ds(1024 * p + 16 * j, 16)]
            k3 = kk * 3
            for c in range(3):
                klist3[pl.ds(3072 * p + 1024 * c + 16 * j, 16)] = k3 + c
            return 0

        lax.fori_loop(0, vpsb, _exp, 0)

    def _gather(p):
        for c in range(3):
            _g_dma(p, c).start()

    def _gwait(p):
        for c in range(3):
            _g_dma(p, c).wait()

    def _patch(p):
        cnt = cnts[p]
        vb = jnp.full((16,), 3072 * p, jnp.int32)
        lb = jnp.full((16,), 1024 * p, jnp.int32)

        def _pstep(t, _):
            jv = iota + 16 * t
            mv = jv < cnt
            jc = jnp.minimum(jv, sbr - 1)
            pos = plsc.load_gather(plist, [lb + jc], mask=mv)
            posc = jnp.minimum(jnp.maximum(pos, 0), sbr - 1)
            p3 = posc * 3
            for c in range(3):
                x = plsc.load_gather(vals, [vb + 1024 * c + jc], mask=mv)
                plsc.store_scatter(inbuf, [vb + p3 + c], x, mask=mv)
            return 0

        lax.fori_loop(0, (cnt + 15) // 16, _pstep, 0)

    nsb = nstrips // 8
    rem = nstrips - nsb * 8

    @pl.when(nsb > 0)
    def _():
        _in_dma(0, 0).start()

    def _sb_pair(q, _):
        for p in range(2):
            s = 2 * q + p

            @pl.when(s < nsb)
            def _():
                _in_dma(s, p).wait()
                _extract(s, p)
                _gather(p)
                @pl.when((s + 1 < nsb) & (s >= 1))
                def _():
                    _out_dma(s - 1, 1 - p).wait()
                @pl.when(s + 1 < nsb)
                def _():
                    _in_dma(s + 1, 1 - p).start()
                _gwait(p)
                _patch(p)
                _out_dma(s, p).start()
        return 0

    lax.fori_loop(0, (nsb + 1) // 2, _sb_pair, 0)
    for p in range(2):
        @pl.when(nsb >= p + 1)
        def _():
            _out_dma(0, p).wait()

    def _tail(j, _):
        s0 = nsb * 8 + j
        cp_in = pltpu.make_async_copy(
            in_hbm.at[pl.ds(3 * (base + s0 * strip), 3 * strip)],
            inbuf.at[pl.ds(0, 3 * strip)], in_sem.at[0])
        cp_in.start()
        cp_in.wait()

        def _pre(jx, _):
            klist[pl.ds(16 * jx, 16)] = padv + 16 * jx
            return 0

        lax.fori_loop(0, strip // 16, _pre, 0)
        srow = s0 * strip

        def _ex(jx, off):
            t = ticket[pl.ds(srow + 16 * jx, 16)]
            msk = t > 0
            plsc.store_compressed(klist.at[pl.ds(off, 16)], t - 1, mask=msk)
            plsc.store_compressed(plist.at[pl.ds(off, 16)], iota + 16 * jx,
                                  mask=msk)
            return off + jnp.sum(msk.astype(jnp.int32))

        cnt = lax.fori_loop(0, strip // 16, _ex, jnp.int32(0))

        def _exp(jx, _):
            kk = klist[pl.ds(16 * jx, 16)]
            k3 = kk * 3
            for c in range(3):
                klist3[pl.ds(1024 * c + 16 * jx, 16)] = k3 + c
            return 0

        lax.fori_loop(0, strip // 16, _exp, 0)
        for c in range(3):
            pltpu.make_async_copy(
                new_hbm.at[klist3.at[pl.ds(1024 * c, strip)]],
                vals.at[pl.ds(1024 * c, strip)], g_sem.at[0]).start()
        for c in range(3):
            pltpu.make_async_copy(
                new_hbm.at[klist3.at[pl.ds(1024 * c, strip)]],
                vals.at[pl.ds(1024 * c, strip)], g_sem.at[0]).wait()

        def _pstep(t, _):
            jv = iota + 16 * t
            mv = jv < cnt
            jc = jnp.minimum(jv, strip - 1)
            pos = plsc.load_gather(plist, [jc], mask=mv)
            posc = jnp.minimum(jnp.maximum(pos, 0), strip - 1)
            p3 = posc * 3
            for c in range(3):
                x = plsc.load_gather(vals, [1024 * c + jc], mask=mv)
                plsc.store_scatter(inbuf, [p3 + c], x, mask=mv)
            return 0

        lax.fori_loop(0, (cnt + 15) // 16, _pstep, 0)
        cp_out = pltpu.make_async_copy(
            inbuf.at[pl.ds(0, 3 * strip)],
            out_hbm.at[pl.ds(3 * (base + s0 * strip), 3 * strip)],
            out_sem.at[0])
        cp_out.start()
        cp_out.wait()
        return 0

    lax.fori_loop(0, rem, _tail, 0)


def _scatter_body(ctl_in, gau_in, new_ctl, new_gau, ci, gi,
                  ctl_out, gau_out, ticket, idxb, inbuf, vals, klist, klist3,
                  plist, cnts, scan_sem, in_sem, g_sem, out_sem):
    w = lax.axis_index("s") * 2 + lax.axis_index("c")
    common = (ticket, idxb, inbuf, vals, klist, klist3, plist, cnts,
              scan_sem, in_sem, g_sem, out_sem)
    _scatter_pass(w, ci, new_ctl, ctl_in, ctl_out, *common,
                  n_chunks=KC // SCAN_CHUNK, base_rows_std=0,
                  rows_std=6240, rows_last=6560, strip=32)
    def _gau_half(h, _):
        _scatter_pass(w, gi, new_gau, gau_in, gau_out, *common,
                      n_chunks=KG // SCAN_CHUNK,
                      base_rows_std=h * 2000000,
                      rows_std=62464, rows_last=63616, strip=128)
        return 0

    lax.fori_loop(0, 2, _gau_half, 0)


def _mean_table(w, table_hbm, idx_hbm, stg, rows, idx3, outv, stg_sem, g_sem,
                *, nblk, extra128, extra32, out_off, partials):
    ibase = w * (nblk * 1024)
    rem_base = NW * (nblk * 1024)
    iota = _IOTA()
    accs = [jnp.zeros((16,), jnp.float32) for _ in range(3)]

    def _stg_dma(blk, slot, size):
        return pltpu.make_async_copy(
            idx_hbm.at[pl.ds(ibase + blk * 1024, size)],
            stg.at[pl.ds(1024 * slot, size)], stg_sem.at[slot])

    def _expand(jj, stg_slot):
        for r in range(8):
            kk = stg[pl.ds(1024 * stg_slot + 128 * jj + 16 * r, 16)]
            k3 = kk * 3
            for c in range(3):
                idx3[pl.ds(384 * jj + 128 * c + 16 * r, 16)] = k3 + c

    def _g_dma(jj, c):
        return pltpu.make_async_copy(
            table_hbm.at[idx3.at[pl.ds(384 * jj + 128 * c, 128)]],
            rows.at[pl.ds(384 * jj + 128 * c, 128)], g_sem.at[jj])

    def _fire(jj, stg_slot):
        _expand(jj, stg_slot)
        for c in range(3):
            _g_dma(jj, c).start()

    def _drain(jj):
        for c in range(3):
            _g_dma(jj, c).wait()

    def _acc_chunk(accs, jj):
        for rv in range(8):
            for c in range(3):
                x = rows[pl.ds(384 * jj + 128 * c + 16 * rv, 16)]
                accs[c] = accs[c] + x
        return accs

    _stg_dma(0, 0, 1024).start()
    _stg_dma(0, 0, 1024).wait()
    for jj in range(8):
        _fire(jj, 0)
    _stg_dma(1, 1, 1024).start()

    def _pair(bp, accs):
        accs = list(accs)
        blk_e = 2 * bp
        _stg_dma(blk_e + 1, 1, 1024).wait()
        for jj in range(8):
            _drain(jj)
            accs = _acc_chunk(accs, jj)
            _fire(jj, 1)
        @pl.when(blk_e + 2 < nblk)
        def _():
            _stg_dma(blk_e + 2, 0, 1024).start()
            _stg_dma(blk_e + 2, 0, 1024).wait()
        @pl.when(blk_e + 3 < nblk)
        def _():
            _stg_dma(blk_e + 3, 1, 1024).start()
        for jj in range(8):
            _drain(jj)
            accs = _acc_chunk(accs, jj)
            @pl.when(blk_e + 2 < nblk)
            def _():
                _fire(jj, 0)
        return tuple(accs)

    accs = list(lax.fori_loop(0, nblk // 2, _pair, tuple(accs)))
    for jj in range(8):
        _drain(jj)
        accs = _acc_chunk(accs, jj)

    sums = [jnp.sum(a) for a in accs]
    vec = jnp.where(iota == 0, jnp.full((16,), sums[0], jnp.float32),
          jnp.where(iota == 1, jnp.full((16,), sums[1], jnp.float32),
          jnp.where(iota == 2, jnp.full((16,), sums[2], jnp.float32),
                    jnp.zeros((16,), jnp.float32))))
    outv[...] = vec

    @pl.when(w == NW - 1)
    def _():
        exaccs = [jnp.zeros((16,), jnp.float32) for _ in range(3)]

        def _extra_body(j, carry):
            cp = pltpu.make_async_copy(
                idx_hbm.at[pl.ds(rem_base + j * 128, 128)],
                stg.at[pl.ds(0, 128)], stg_sem.at[0])
            cp.start()
            cp.wait()
            _fire(0, 0)
            _drain(0)
            return tuple(_acc_chunk(list(carry), 0))

        ex = list(lax.fori_loop(0, extra128, _extra_body, tuple(exaccs)))
        if extra32:
            base32 = rem_base + extra128 * 128
            cp = pltpu.make_async_copy(
                idx_hbm.at[pl.ds(base32, 32)], stg.at[pl.ds(0, 32)],
                stg_sem.at[0])
            cp.start()
            cp.wait()
            for r in range(2):
                kk = stg[pl.ds(16 * r, 16)]
                k3 = kk * 3
                for c in range(3):
                    idx3[pl.ds(32 * c + 16 * r, 16)] = k3 + c
            for c in range(3):
                pltpu.make_async_copy(
                    table_hbm.at[idx3.at[pl.ds(32 * c, 32)]],
                    rows.at[pl.ds(32 * c, 32)], g_sem.at[0]).start()
            for c in range(3):
                pltpu.make_async_copy(
                    table_hbm.at[idx3.at[pl.ds(32 * c, 32)]],
                    rows.at[pl.ds(32 * c, 32)], g_sem.at[0]).wait()
            for rv in range(2):
                for c in range(3):
                    x = rows[pl.ds(32 * c + 16 * rv, 16)]
                    ex[c] = ex[c] + x
        exsums = [jnp.sum(a) for a in ex]
        exvec = jnp.where(iota == 0, jnp.full((16,), exsums[0], jnp.float32),
                jnp.where(iota == 1, jnp.full((16,), exsums[1], jnp.float32),
                jnp.where(iota == 2, jnp.full((16,), exsums[2], jnp.float32),
                          jnp.zeros((16,), jnp.float32))))
        outv[...] = outv[...] + exvec

    pltpu.sync_copy(outv, partials.at[w, pl.ds(out_off, 16)])


def _mean_body(ctl_in, gau_in, ci, gi, partials, stg, rows, idx3,
               outv, stg_sem, g_sem):
    w = lax.axis_index("s") * 2 + lax.axis_index("c")
    _mean_table(w, ctl_in, ci, stg, rows, idx3, outv, stg_sem, g_sem,
                nblk=MB_CTL, extra128=13, extra32=True, out_off=0,
                partials=partials)
    _mean_table(w, gau_in, gi, stg, rows, idx3, outv, stg_sem, g_sem,
                nblk=MB_GAU, extra128=9, extra32=False, out_off=16,
                partials=partials)


def _reduce_body(p_ref, o_ref):
    s = jnp.sum(p_ref[...], axis=0, keepdims=True)
    scale = jnp.concatenate([
        jnp.full((1, 3), 0.5 / KC, jnp.float32),
        jnp.zeros((1, 13), jnp.float32),
        jnp.full((1, 3), 0.5 / KG, jnp.float32),
        jnp.zeros((1, 13), jnp.float32),
    ], axis=1)
    o_ref[...] = jnp.pad(s * scale, ((0, 7), (0, 96)))


@jax.jit
def kernel(control_xyz, gaussian_xyz, new_control_xyz, new_gaussian_xyz,
           control_indices, gaussian_indices):
    ctl_flat = control_xyz.reshape(-1) + 0.0
    gau_flat = gaussian_xyz.reshape(-1) + 0.0
    new_ctl_flat = new_control_xyz.reshape(-1) + 0.0
    new_gau_flat = new_gaussian_xyz.reshape(-1) + 0.0

    mesh = plsc.VectorSubcoreMesh(core_axis_name="c", subcore_axis_name="s")

    scatter_fn = pl.kernel(
        _scatter_body,
        out_type=[
            jax.ShapeDtypeStruct((3 * M,), jnp.float32),
            jax.ShapeDtypeStruct((3 * N,), jnp.float32),
        ],
        mesh=mesh,
        compiler_params=pltpu.CompilerParams(needs_layout_passes=False),
        scratch_types=[
            pltpu.VMEM((63616,), jnp.int32),
            pltpu.VMEM((2 * SCAN_CHUNK,), jnp.int32),
            pltpu.VMEM((2 * 3072,), jnp.float32),
            pltpu.VMEM((2 * 3072,), jnp.float32),
            pltpu.VMEM((2 * 1024,), jnp.int32),
            pltpu.VMEM((2 * 3072,), jnp.int32),
            pltpu.VMEM((2 * 1024,), jnp.int32),
            pltpu.SMEM((2,), jnp.int32),
            pltpu.SemaphoreType.DMA((2,)),
            pltpu.SemaphoreType.DMA((2,)),
            pltpu.SemaphoreType.DMA((2,)),
            pltpu.SemaphoreType.DMA((2,)),
        ],
    )
    updated_ctl_flat, updated_gau_flat = scatter_fn(
        ctl_flat, gau_flat, new_ctl_flat, new_gau_flat, control_indices,
        gaussian_indices)

    mean_fn = pl.kernel(
        _mean_body,
        out_type=jax.ShapeDtypeStruct((NW, 32), jnp.float32),
        mesh=mesh,
        compiler_params=pltpu.CompilerParams(needs_layout_passes=False),
        scratch_types=[
            pltpu.VMEM((2 * 1024,), jnp.int32),
            pltpu.VMEM((8 * 384,), jnp.float32),
            pltpu.VMEM((8 * 384,), jnp.int32),
            pltpu.VMEM((16,), jnp.float32),
            pltpu.SemaphoreType.DMA((2,)),
            pltpu.SemaphoreType.DMA((8,)),
        ],
    )
    partials = mean_fn(ctl_flat, gau_flat, control_indices,
                       gaussian_indices)

    red = pl.pallas_call(
        _reduce_body,
        out_shape=jax.ShapeDtypeStruct((8, 128), jnp.float32),
    )(partials)
    center = red[0, 0:3] + red[0, 16:19]

    return (center, updated_ctl_flat.reshape(M, 3) + 0.0,
            updated_gau_flat.reshape(N, 3) + 0.0)
